# baseline (speedup 1.0000x reference)
import jax
import jax.numpy as jnp
from jax import lax
from jax.experimental import pallas as pl
from jax.experimental.pallas import tpu as pltpu
from jax.experimental.pallas import tpu_sc as plsc

NC, NS, L = 2, 16, 16
NW = NC * NS
B, S, K = 4, 128, 128
ROWS = B * S * S
PAIRS = B * S
PAIRS_PER_W = PAIRS // NW
CH = K // L


def _body(in_hbm, out_hbm, idx_v, gbuf, rbuf, sem, osem):
    wid = lax.axis_index("s") * NC + lax.axis_index("c")
    lanes = lax.iota(jnp.int32, L)
    rev_pos = [(CH - 1 - c) * L + (L - 1) - lanes for c in range(CH)]

    def fetch(t, slot):
        p = wid * PAIRS_PER_W + t
        b = p // S
        i = p - b * S
        base = b * (S * S) + i
        for c in range(CH):
            idx_v[slot, pl.ds(c * L, L)] = base + S * (c * L + lanes)
        return pltpu.async_copy(in_hbm.at[idx_v.at[slot]], gbuf.at[slot], sem.at[slot])

    def drain(t, slot, dma):
        p = wid * PAIRS_PER_W + t
        dma.wait()

        @plsc.parallel_loop(0, S, unroll=2)
        def _(j):
            for c in range(CH):
                rbuf[slot, j, pl.ds((CH - 1 - c) * L, L)] = lax.rev(
                    gbuf[slot, j, pl.ds(c * L, L)], (0,)
                )
        return pltpu.async_copy(rbuf.at[slot], out_hbm.at[pl.ds(p * S, S)], osem.at[slot])

    dma = fetch(0, 0)
    st = [None, None]
    for t in range(PAIRS_PER_W):
        nxt = fetch(t + 1, (t + 1) % 2) if t + 1 < PAIRS_PER_W else None
        slot = t % 2
        if st[slot] is not None:
            st[slot].wait()
        st[slot] = drain(t, slot, dma)
        dma = nxt
    st[0].wait()
    st[1].wait()


@jax.jit
def kernel(inputs):
    rows = inputs.reshape(ROWS, K)
    out = pl.kernel(
        _body,
        out_type=jax.ShapeDtypeStruct((ROWS, K), jnp.float32),
        mesh=plsc.VectorSubcoreMesh(core_axis_name="c", subcore_axis_name="s"),
        compiler_params=pltpu.CompilerParams(needs_layout_passes=False),
        scratch_types=[
            pltpu.VMEM((2, K), jnp.int32),
            pltpu.VMEM((2, S, K), jnp.float32),
            pltpu.VMEM((2, S, K), jnp.float32),
            pltpu.SemaphoreType.DMA((2,)),
            pltpu.SemaphoreType.DMA((2,)),
        ],
    )(rows)
    return out.reshape(B, S, S, S, 1)


# speedup vs baseline: 1.0768x; 1.0768x over previous
import jax
import jax.numpy as jnp
from jax import lax
from jax.experimental import pallas as pl
from jax.experimental.pallas import tpu as pltpu
from jax.experimental.pallas import tpu_sc as plsc

NC, NS, L = 2, 16, 16
NW = NC * NS
B, S, K = 4, 128, 128
ROWS = B * S * S
PAIRS = B * S
PAIRS_PER_W = PAIRS // NW
CH = K // L


def _body(in_hbm, out_hbm, idx_v, gbuf, rbuf, sem, osem):
    wid = lax.axis_index("s") * NC + lax.axis_index("c")
    lanes = lax.iota(jnp.int32, L)
    rev_pos = [(CH - 1 - c) * L + (L - 1) - lanes for c in range(CH)]

    def fetch(t, slot):
        p = wid * PAIRS_PER_W + t
        b = p // S
        i = p - b * S
        base = b * (S * S) + i
        for c in range(CH):
            idx_v[slot, pl.ds(c * L, L)] = base + S * (c * L + lanes)
        return pltpu.async_copy(in_hbm.at[idx_v.at[slot]], gbuf.at[slot], sem.at[slot])

    def wait_gather(slot):
        pltpu.make_async_copy(in_hbm.at[pl.ds(0, S)], gbuf.at[slot], sem.at[slot]).wait()

    def wait_store(slot):
        pltpu.make_async_copy(rbuf.at[slot], out_hbm.at[pl.ds(0, S)], osem.at[slot]).wait()

    def half(it, t, slot):
        wait_gather(slot)

        @pl.when(it >= 1)
        def _():
            wait_store(slot)

        @plsc.parallel_loop(0, S)
        def _(j):
            for c in range(CH):
                rbuf[slot, j, pl.ds((CH - 1 - c) * L, L)] = lax.rev(
                    gbuf[slot, j, pl.ds(c * L, L)], (0,)
                )

        p = wid * PAIRS_PER_W + t
        pltpu.make_async_copy(
            rbuf.at[slot], out_hbm.at[pl.ds(p * S, S)], osem.at[slot]
        ).start()

        @pl.when(t + 2 < PAIRS_PER_W)
        def _():
            fetch(t + 2, slot)

    fetch(0, 0)
    fetch(1, 1)

    def step(it, carry):
        half(it, 2 * it, 0)
        half(it, 2 * it + 1, 1)
        return carry

    lax.fori_loop(0, PAIRS_PER_W // 2, step, 0)
    wait_store(0)
    wait_store(1)


@jax.jit
def kernel(inputs):
    rows = inputs.reshape(ROWS, K)
    out = pl.kernel(
        _body,
        out_type=jax.ShapeDtypeStruct((ROWS, K), jnp.float32),
        mesh=plsc.VectorSubcoreMesh(core_axis_name="c", subcore_axis_name="s"),
        compiler_params=pltpu.CompilerParams(needs_layout_passes=False),
        scratch_types=[
            pltpu.VMEM((2, K), jnp.int32),
            pltpu.VMEM((2, S, K), jnp.float32),
            pltpu.VMEM((2, S, K), jnp.float32),
            pltpu.SemaphoreType.DMA((2,)),
            pltpu.SemaphoreType.DMA((2,)),
        ],
    )(rows)
    return out.reshape(B, S, S, S, 1)


# speedup vs baseline: 1.0798x; 1.0027x over previous
import jax
import jax.numpy as jnp
from jax import lax
from jax.experimental import pallas as pl
from jax.experimental.pallas import tpu as pltpu
from jax.experimental.pallas import tpu_sc as plsc

NC, NS, L = 2, 16, 16
NW = NC * NS
B, S, K = 4, 128, 128
ROWS = B * S * S
PAIRS = B * S
PAIRS_PER_W = PAIRS // NW
CH = K // L


def _body(in_hbm, out_hbm, idx_v, gbuf, rbuf, sem, osem):
    wid = lax.axis_index("s") * NC + lax.axis_index("c")
    lanes = lax.iota(jnp.int32, L)

    def fetch(t, slot):
        p = wid * PAIRS_PER_W + t
        b = p // S
        i = p - b * S
        base = b * (S * S) + i
        for c in range(CH):
            idx_v[slot, pl.ds(c * L, L)] = base + S * (c * L + lanes)
        return pltpu.async_copy(in_hbm.at[idx_v.at[slot]], gbuf.at[slot], sem.at[slot])

    def wait_gather(slot):
        pltpu.make_async_copy(in_hbm.at[pl.ds(0, S)], gbuf.at[slot], sem.at[slot]).wait()

    def wait_store(slot):
        pltpu.make_async_copy(rbuf.at[slot], out_hbm.at[pl.ds(0, S)], osem.at[slot]).wait()

    def half(it, t, slot):
        wait_gather(slot)

        @pl.when(it >= 1)
        def _():
            wait_store(slot)

        @plsc.parallel_loop(0, S)
        def _(j):
            for c in range(CH):
                rbuf[slot, j, pl.ds((CH - 1 - c) * L, L)] = lax.rev(
                    gbuf[slot, j, pl.ds(c * L, L)], (0,)
                )

        p = wid * PAIRS_PER_W + t
        pltpu.make_async_copy(
            rbuf.at[slot], out_hbm.at[pl.ds(p * S, S)], osem.at[slot]
        ).start()

        @pl.when(t + 2 < PAIRS_PER_W)
        def _():
            fetch(t + 2, slot)

    fetch(0, 0)
    fetch(1, 1)

    def step(it, carry):
        half(it, 2 * it, 0)
        half(it, 2 * it + 1, 1)
        return carry

    lax.fori_loop(0, PAIRS_PER_W // 2, step, 0)
    wait_store(0)
    wait_store(1)


@jax.jit
def kernel(inputs):
    rows = inputs.reshape(ROWS, K)
    out = pl.kernel(
        _body,
        out_type=jax.ShapeDtypeStruct((ROWS, K), jnp.float32),
        mesh=plsc.VectorSubcoreMesh(core_axis_name="c", subcore_axis_name="s"),
        compiler_params=pltpu.CompilerParams(needs_layout_passes=False),
        scratch_types=[
            pltpu.VMEM((2, K), jnp.int32),
            pltpu.VMEM((2, S, K), jnp.float32),
            pltpu.VMEM((2, S, K), jnp.float32),
            pltpu.SemaphoreType.DMA((2,)),
            pltpu.SemaphoreType.DMA((2,)),
        ],
    )(rows)
    return out.reshape(B, S, S, S, 1)
